# baseline (speedup 1.0000x reference)
import jax
import jax.numpy as jnp
from jax import lax
from jax.experimental import pallas as pl
from jax.experimental.pallas import tpu as pltpu
from jax.experimental.pallas import tpu_sc as plsc

B = 16384
D = 64
NC, NS, L = 2, 16, 16
NW = NC * NS
BPW = B // NW
CH = 128
NCH = BPW // CH
NG = BPW // L
PAD = L + 1

_MU = 3.53


def _svd_body(uid_hbm, iid_hbm, ub_hbm, ib_hbm, uf_hbm, if_hbm, out_hbm,
              uidx_v, iidx_v, pu_v, qi_v, bu_v, bi_v, res_v, scr_v, sem):
  wid = lax.axis_index("s") * NC + lax.axis_index("c")
  base = wid * BPW

  for c in range(NCH):
    pltpu.sync_copy(uid_hbm.at[pl.ds(base + c * CH, CH)], uidx_v.at[c])
    pltpu.sync_copy(iid_hbm.at[pl.ds(base + c * CH, CH)], iidx_v.at[c])

  handles = []
  for c in range(NCH):
    sl = pl.ds(c * CH, CH)
    handles.append(pltpu.async_copy(uf_hbm.at[uidx_v.at[c]], pu_v.at[sl], sem))
    handles.append(pltpu.async_copy(if_hbm.at[iidx_v.at[c]], qi_v.at[sl], sem))
    handles.append(pltpu.async_copy(ub_hbm.at[uidx_v.at[c]], bu_v.at[sl], sem))
    handles.append(pltpu.async_copy(ib_hbm.at[iidx_v.at[c]], bi_v.at[sl], sem))
  for h in handles:
    h.wait()

  lane = lax.iota(jnp.int32, L)
  col_idx = lane * PAD

  def group_body(g, carry):
    row0 = g * L
    for rr in range(L):
      r = row0 + rr
      acc = pu_v[r, pl.ds(0, L)] * qi_v[r, pl.ds(0, L)]
      for k in range(1, D // L):
        acc = acc + pu_v[r, pl.ds(k * L, L)] * qi_v[r, pl.ds(k * L, L)]
      scr_v[pl.ds(rr * PAD, L)] = acc
    dots0 = plsc.load_gather(scr_v, [col_idx])
    dots1 = plsc.load_gather(scr_v, [col_idx + 1])
    dots2 = plsc.load_gather(scr_v, [col_idx + 2])
    dots3 = plsc.load_gather(scr_v, [col_idx + 3])
    for l in range(4, L, 4):
      dots0 = dots0 + plsc.load_gather(scr_v, [col_idx + l])
      dots1 = dots1 + plsc.load_gather(scr_v, [col_idx + l + 1])
      dots2 = dots2 + plsc.load_gather(scr_v, [col_idx + l + 2])
      dots3 = dots3 + plsc.load_gather(scr_v, [col_idx + l + 3])
    dots = (dots0 + dots1) + (dots2 + dots3)
    rating = (jnp.float32(_MU) + bu_v[pl.ds(row0, L)] + bi_v[pl.ds(row0, L)]
              + dots)
    rating = jnp.minimum(jnp.maximum(rating, jnp.float32(1.0)),
                         jnp.float32(5.0))
    res_v[pl.ds(row0, L)] = rating
    return carry

  lax.fori_loop(0, NG, group_body, 0)
  pltpu.sync_copy(res_v, out_hbm.at[pl.ds(base, BPW)])


@jax.jit
def kernel(user_ids, item_ids, user_bias, item_bias, user_factors,
           item_factors):
  mesh = plsc.VectorSubcoreMesh(core_axis_name="c", subcore_axis_name="s")
  run = pl.kernel(
      _svd_body,
      out_type=jax.ShapeDtypeStruct((B,), jnp.float32),
      mesh=mesh,
      compiler_params=pltpu.CompilerParams(needs_layout_passes=False,
                                           use_tc_tiling_on_sc=False),
      scratch_types=[
          pltpu.VMEM((NCH, CH), jnp.int32),
          pltpu.VMEM((NCH, CH), jnp.int32),
          pltpu.VMEM((BPW, D), jnp.float32),
          pltpu.VMEM((BPW, D), jnp.float32),
          pltpu.VMEM((BPW,), jnp.float32),
          pltpu.VMEM((BPW,), jnp.float32),
          pltpu.VMEM((BPW,), jnp.float32),
          pltpu.VMEM((L * PAD,), jnp.float32),
          pltpu.SemaphoreType.DMA,
      ],
  )
  return run(user_ids.astype(jnp.int32), item_ids.astype(jnp.int32),
             user_bias.reshape(-1), item_bias.reshape(-1),
             user_factors, item_factors)
